# Initial kernel scaffold; baseline (speedup 1.0000x reference)
import jax
import jax.numpy as jnp
from jax.experimental import pallas as pl


def kernel(node_feats, W, b):
    raise NotImplementedError("write your pallas kernel here")


# speedup vs baseline: 7.1216x; 7.1216x over previous
import jax
import jax.numpy as jnp
import numpy as np
from jax.experimental import pallas as pl
from jax.experimental.pallas import tpu as pltpu

_OUT = 32
_PROJ = _OUT * (_OUT + 1) // 2
_FLAT = _OUT * _OUT


def _sym_perm() -> np.ndarray:
    rows, cols = np.triu_indices(_OUT)
    m = np.zeros((_OUT, _OUT), dtype=np.int32)
    m[rows, cols] = np.arange(_PROJ, dtype=np.int32)
    m[cols, rows] = np.arange(_PROJ, dtype=np.int32)
    return m.reshape(-1)


_G = _sym_perm()


def _proj_kernel(x_ref, w_ref, b_ref, o_ref):
    o_ref[...] = (
        jnp.dot(x_ref[...], w_ref[...], preferred_element_type=jnp.float32)
        + b_ref[...]
    )


def kernel(node_feats, W, b):
    n, d = node_feats.shape
    w2 = W[_G].T.astype(jnp.float32)
    b2 = b[_G][None, :].astype(jnp.float32)

    bn = 2000
    if n % bn != 0:
        bn = next(s for s in (1000, 500, 200, 100, 50, 25, 8, 1) if n % s == 0)
    grid = n // bn

    out = pl.pallas_call(
        _proj_kernel,
        grid=(grid,),
        in_specs=[
            pl.BlockSpec((bn, d), lambda i: (i, 0)),
            pl.BlockSpec((d, _FLAT), lambda i: (0, 0)),
            pl.BlockSpec((1, _FLAT), lambda i: (0, 0)),
        ],
        out_specs=pl.BlockSpec((bn, _FLAT), lambda i: (i, 0)),
        out_shape=jax.ShapeDtypeStruct((n, _FLAT), jnp.float32),
        compiler_params=pltpu.CompilerParams(
            dimension_semantics=("parallel",)
        ),
    )(node_feats, w2, b2)
    return out.reshape(n, _OUT, _OUT)
